# baseline (speedup 1.0000x reference)
import jax
import jax.numpy as jnp
from jax import lax
from jax.experimental import pallas as pl
from jax.experimental.pallas import tpu as pltpu
from jax.experimental.pallas import tpu_sc as plsc

DIM = 1024
T = 20
LANES = 16
CHUNKS = DIM // LANES
NEWTON_ITERS = 3
EPS = 1e-5

_GATHER_DN = lax.GatherDimensionNumbers(
    offset_dims=(), collapsed_slice_dims=(0,), start_index_map=(0,))


def _lane_shuffle(v, idx):
    return lax.gather(v, idx[:, None], _GATHER_DN, slice_sizes=(1,),
                      mode=lax.GatherScatterMode.PROMISE_IN_BOUNDS)


def _lane_allreduce_sum(v):
    lanes = lax.iota(jnp.int32, LANES)
    for shift in (1, 2, 4, 8):
        v = v + _lane_shuffle(v, lanes ^ shift)
    return v


def _rsqrt16(a):
    xi = lax.bitcast_convert_type(a, jnp.int32)
    yi = jnp.int32(0x5F3759DF) - (xi >> 1)
    y = lax.bitcast_convert_type(yi, jnp.float32)
    half = a * 0.5
    for _ in range(NEWTON_ITERS):
        y = y * (1.5 - half * y * y)
    return y


def _make_sc_kernel(num_bags, bags_per_w):
    mesh = plsc.VectorSubcoreMesh(core_axis_name="c", subcore_axis_name="s")
    nc = mesh.num_cores

    def run(idx, table):
        @pl.kernel(
            out_type=jax.ShapeDtypeStruct((num_bags, DIM), jnp.float32),
            mesh=mesh,
            scratch_types=[
                pltpu.VMEM((bags_per_w, T), jnp.int32),
                pltpu.VMEM((2, T, DIM), jnp.float32),
                pltpu.VMEM((2, DIM), jnp.float32),
                pltpu.SemaphoreType.DMA,
                pltpu.SemaphoreType.DMA,
                pltpu.SemaphoreType.DMA,
                pltpu.SemaphoreType.DMA,
            ],
            compiler_params=pltpu.CompilerParams(use_tc_tiling_on_sc=False),
        )
        def body(idx_hbm, table_hbm, out_hbm, idx_v, rows_v, row_v,
                 gsem0, gsem1, osem0, osem1):
            wid = lax.axis_index("s") * nc + lax.axis_index("c")
            base = wid * bags_per_w
            pltpu.sync_copy(idx_hbm.at[pl.ds(base, bags_per_w)], idx_v)
            zeros = jnp.zeros((LANES,), jnp.float32)
            gsems = (gsem0, gsem1)
            osems = (osem0, osem1)

            pltpu.async_copy(table_hbm.at[idx_v.at[0]], rows_v.at[0], gsem0)

            def do_bag(j, slot):
                pltpu.make_async_copy(
                    table_hbm.at[idx_v.at[j]], rows_v.at[slot],
                    gsems[slot]).wait()

                @pl.when(j >= 2)
                def _():
                    pltpu.make_async_copy(
                        row_v.at[slot], out_hbm.at[base + j - 2],
                        osems[slot]).wait()

                @pl.when(j + 1 < bags_per_w)
                def _():
                    pltpu.async_copy(
                        table_hbm.at[idx_v.at[j + 1]],
                        rows_v.at[1 - slot], gsems[1 - slot])

                def chunk_body(c, carry):
                    vsum, vsq = carry
                    vals = [rows_v[slot, t, pl.ds(c * LANES, LANES)]
                            for t in range(T)]
                    while len(vals) > 1:
                        nxt = [vals[k] + vals[k + 1]
                               for k in range(0, len(vals) - 1, 2)]
                        if len(vals) % 2:
                            nxt[-1] = nxt[-1] + vals[-1]
                        vals = nxt
                    s = vals[0]
                    row_v[slot, pl.ds(c * LANES, LANES)] = s
                    return (vsum + s, vsq + s * s)

                vsum, vsq = lax.fori_loop(
                    0, CHUNKS, chunk_body, (zeros, zeros), unroll=2)
                mean = _lane_allreduce_sum(vsum) * (1.0 / DIM)
                ex2 = _lane_allreduce_sum(vsq) * (1.0 / DIM)
                rstd = _rsqrt16(ex2 - mean * mean + EPS)

                def norm_body(c, _):
                    v = row_v[slot, pl.ds(c * LANES, LANES)]
                    row_v[slot, pl.ds(c * LANES, LANES)] = (v - mean) * rstd
                    return 0

                lax.fori_loop(0, CHUNKS, norm_body, 0)
                pltpu.async_copy(row_v.at[slot], out_hbm.at[base + j],
                                 osems[slot])

            @pl.loop(0, bags_per_w, step=2)
            def _(i):
                for b in range(2):
                    do_bag(i + b, b)

            for j, slot in ((bags_per_w - 2, 0), (bags_per_w - 1, 1)):
                pltpu.make_async_copy(
                    row_v.at[slot], out_hbm.at[base + j], osems[slot]).wait()

        return body(idx, table)

    return run


_NUM_WORKERS = 32
_sc_run = None


def kernel(x, table):
    global _sc_run
    B, F, t = x.shape
    num_bags = B * F
    if _sc_run is None:
        _sc_run = _make_sc_kernel(num_bags, num_bags // _NUM_WORKERS)
    idx = x.reshape(num_bags, t)
    out = _sc_run(idx, table)
    return out.reshape(B, F, table.shape[1])


# speedup vs baseline: 1.2192x; 1.2192x over previous
import jax
import jax.numpy as jnp
from jax import lax
from jax.experimental import pallas as pl
from jax.experimental.pallas import tpu as pltpu
from jax.experimental.pallas import tpu_sc as plsc

DIM = 1024
T = 20
LANES = 16
CHUNKS = DIM // LANES
NEWTON_ITERS = 3
EPS = 1e-5
NSLOT = 4

_GATHER_DN = lax.GatherDimensionNumbers(
    offset_dims=(), collapsed_slice_dims=(0,), start_index_map=(0,))


def _lane_shuffle(v, idx):
    return lax.gather(v, idx[:, None], _GATHER_DN, slice_sizes=(1,),
                      mode=lax.GatherScatterMode.PROMISE_IN_BOUNDS)


def _lane_allreduce_sum(v):
    lanes = lax.iota(jnp.int32, LANES)
    for shift in (1, 2, 4, 8):
        v = v + _lane_shuffle(v, lanes ^ shift)
    return v


def _rsqrt16(a):
    xi = lax.bitcast_convert_type(a, jnp.int32)
    yi = jnp.int32(0x5F3759DF) - (xi >> 1)
    y = lax.bitcast_convert_type(yi, jnp.float32)
    half = a * 0.5
    for _ in range(NEWTON_ITERS):
        y = y * (1.5 - half * y * y)
    return y


def _make_sc_kernel(num_bags, bags_per_w):
    mesh = plsc.VectorSubcoreMesh(core_axis_name="c", subcore_axis_name="s")
    nc = mesh.num_cores

    def run(idx, table):
        @pl.kernel(
            out_type=jax.ShapeDtypeStruct((num_bags, DIM), jnp.float32),
            mesh=mesh,
            scratch_types=[
                pltpu.VMEM((bags_per_w, T), jnp.int32),
                pltpu.VMEM((NSLOT, T, DIM), jnp.float32),
                pltpu.VMEM((NSLOT, DIM), jnp.float32),
            ] + [pltpu.SemaphoreType.DMA] * (2 * NSLOT),
            compiler_params=pltpu.CompilerParams(use_tc_tiling_on_sc=False),
        )
        def body(idx_hbm, table_hbm, out_hbm, idx_v, rows_v, row_v, *sems):
            gsems = sems[:NSLOT]
            osems = sems[NSLOT:]
            wid = lax.axis_index("s") * nc + lax.axis_index("c")
            base = wid * bags_per_w
            pltpu.sync_copy(idx_hbm.at[pl.ds(base, bags_per_w)], idx_v)
            zeros = jnp.zeros((LANES,), jnp.float32)

            for s in range(NSLOT - 1):
                pltpu.async_copy(
                    table_hbm.at[idx_v.at[s]], rows_v.at[s], gsems[s])

            def do_bag(j, slot):
                pltpu.make_async_copy(
                    table_hbm.at[idx_v.at[j]], rows_v.at[slot],
                    gsems[slot]).wait()

                @pl.when(j >= NSLOT)
                def _():
                    pltpu.make_async_copy(
                        row_v.at[slot], out_hbm.at[base + j - NSLOT],
                        osems[slot]).wait()

                nslot = (slot + NSLOT - 1) % NSLOT
                @pl.when(j + NSLOT - 1 < bags_per_w)
                def _():
                    pltpu.async_copy(
                        table_hbm.at[idx_v.at[j + NSLOT - 1]],
                        rows_v.at[nslot], gsems[nslot])

                def chunk_body(c, carry):
                    vsum, vsq = carry
                    vals = [rows_v[slot, t, pl.ds(c * LANES, LANES)]
                            for t in range(T)]
                    while len(vals) > 1:
                        nxt = [vals[k] + vals[k + 1]
                               for k in range(0, len(vals) - 1, 2)]
                        if len(vals) % 2:
                            nxt[-1] = nxt[-1] + vals[-1]
                        vals = nxt
                    s = vals[0]
                    row_v[slot, pl.ds(c * LANES, LANES)] = s
                    return (vsum + s, vsq + s * s)

                vsum, vsq = lax.fori_loop(
                    0, CHUNKS, chunk_body, (zeros, zeros), unroll=2)
                mean = _lane_allreduce_sum(vsum) * (1.0 / DIM)
                ex2 = _lane_allreduce_sum(vsq) * (1.0 / DIM)
                rstd = _rsqrt16(ex2 - mean * mean + EPS)

                def norm_body(c, _):
                    v = row_v[slot, pl.ds(c * LANES, LANES)]
                    row_v[slot, pl.ds(c * LANES, LANES)] = (v - mean) * rstd
                    return 0

                lax.fori_loop(0, CHUNKS, norm_body, 0)
                pltpu.async_copy(row_v.at[slot], out_hbm.at[base + j],
                                 osems[slot])

            @pl.loop(0, bags_per_w, step=NSLOT)
            def _(i):
                for b in range(NSLOT):
                    do_bag(i + b, b)

            for s in range(NSLOT):
                j = bags_per_w - NSLOT + s
                pltpu.make_async_copy(
                    row_v.at[s], out_hbm.at[base + j], osems[s]).wait()

        return body(idx, table)

    return run


_NUM_WORKERS = 32
_sc_run = None


def kernel(x, table):
    global _sc_run
    B, F, t = x.shape
    num_bags = B * F
    if _sc_run is None:
        _sc_run = _make_sc_kernel(num_bags, num_bags // _NUM_WORKERS)
    idx = x.reshape(num_bags, t)
    out = _sc_run(idx, table)
    return out.reshape(B, F, table.shape[1])
